# Initial kernel scaffold; baseline (speedup 1.0000x reference)
import jax
import jax.numpy as jnp
from jax.experimental import pallas as pl


def kernel(x, edge_index, Wl1, bl1, Wr1, Wl2, bl2, Wr2, Wc1, bc1, Wc2, bc2):
    raise NotImplementedError("write your pallas kernel here")


# speedup vs baseline: 2.2502x; 2.2502x over previous
import functools

import jax
import jax.numpy as jnp
from jax import lax
from jax.experimental import pallas as pl
from jax.experimental.pallas import tpu as pltpu
from jax.experimental.pallas import tpu_sc as plsc

N = 10000
E = 320000
D_IN = 128
D_HID = 128
D_OUT = 64
DW = 128

NC = 2
NS = 16
NW = NC * NS
CH = 128
NCHUNK = 80
EPW = NCHUNK * CH
EPAD = NW * EPW
NPAD = 10240
RSTRIPE = NPAD // NS
OB = 128

_F32 = jnp.float32
_HIGH = jax.lax.Precision.HIGHEST


def _dot(a, b):
    return jnp.dot(a, b, preferred_element_type=_F32, precision=_HIGH)


def _tc1_body(x_ref, wl1t_ref, o_ref):
    o_ref[...] = _dot(x_ref[...], wl1t_ref[...])


def _tc2_body(cnt_ref, s1_ref, x_ref, wr1t_ref, bl1_ref, wl2t_ref,
              h1_ref, p2_ref, rinv_ref):
    cnt = (cnt_ref[0] + cnt_ref[1])[:N, :1]
    rinv = 1.0 / jnp.maximum(cnt, 1.0)
    s = (s1_ref[0] + s1_ref[1])[:N]
    mean = s * rinv
    h1 = jnp.maximum(mean + bl1_ref[...] + _dot(x_ref[...], wr1t_ref[...]), 0.0)
    h1_ref[...] = h1
    p2_ref[:, :D_OUT] = _dot(h1, wl2t_ref[...])
    p2_ref[:, D_OUT:] = jnp.zeros((N, DW - D_OUT), _F32)
    rinv_ref[...] = rinv


def _tc3_body(s2_ref, rinv_ref, h1_ref, wr2t_ref, bl2_ref, w1lt_ref,
              w1rt_ref, bc1_ref, ab_ref):
    s = (s2_ref[0] + s2_ref[1])[:N, :D_OUT]
    mean = s * rinv_ref[...]
    h2 = jnp.maximum(mean + bl2_ref[...] + _dot(h1_ref[...], wr2t_ref[...]),
                     0.0)
    ab_ref[:, :D_OUT] = _dot(h2, w1lt_ref[...]) + bc1_ref[...]
    ab_ref[:, D_OUT:] = _dot(h2, w1rt_ref[...])


def _tc4_body(g1_ref, g2_ref, w2t_ref, bc2_ref, o_ref):
    z = jnp.maximum(g1_ref[:, :D_OUT] + g2_ref[:, D_OUT:], 0.0)
    o_ref[...] = _dot(z, w2t_ref[...]) + bc2_ref[...]


_SC_MESH = plsc.VectorSubcoreMesh(
    core_axis_name="c", subcore_axis_name="s", num_cores=NC, num_subcores=NS)


def _fill_vmem(buf, rows, value):
    vec = jnp.full((16,), value, _F32)
    per_row = DW // 16

    def body(i, _):
        buf[i // per_row, pl.ds((i % per_row) * 16, 16)] = vec
        return 0

    lax.fori_loop(0, rows * per_row, body, 0)


def _segsum_body(gather_table, p_hbm, src_hbm, dst_hbm, out_hbm,
                 sidx, didx, rows, acc, sem):
    c = lax.axis_index("c")
    s = lax.axis_index("s")
    wid = s * NC + c
    tb = s * RSTRIPE

    _fill_vmem(rows, OB, 0.0)
    for k in range(RSTRIPE // OB):
        pltpu.sync_copy(rows, acc.at[pl.ds(tb + k * OB, OB)])

    if gather_table:
        pltpu.sync_copy(src_hbm.at[wid], sidx)
    else:
        _fill_vmem(rows, CH, 1.0)
    pltpu.sync_copy(dst_hbm.at[wid], didx)

    plsc.subcore_barrier()

    if gather_table:
        def chunk(j, _):
            pltpu.async_copy(p_hbm.at[sidx.at[j]], rows, sem).wait()
            pltpu.sync_copy(rows, acc.at[didx.at[j]], add=True)
            return 0
    else:
        def chunk(j, _):
            pltpu.sync_copy(rows, acc.at[didx.at[j]], add=True)
            return 0

    lax.fori_loop(0, NCHUNK, chunk, 0)

    plsc.subcore_barrier()

    for k in range(RSTRIPE // OB):
        pltpu.sync_copy(acc.at[pl.ds(tb + k * OB, OB)], rows)
        pltpu.sync_copy(rows, out_hbm.at[c, pl.ds(tb + k * OB, OB)])


_SEG_SCRATCH = [
    pltpu.VMEM((NCHUNK, CH), jnp.int32),
    pltpu.VMEM((NCHUNK, CH), jnp.int32),
    pltpu.VMEM((CH, DW), _F32),
    pltpu.VMEM_SHARED((NPAD, DW), _F32),
    pltpu.SemaphoreType.DMA,
]

_segsum = functools.partial(
    pl.kernel,
    functools.partial(_segsum_body, True),
    out_type=jax.ShapeDtypeStruct((NC, NPAD, DW), _F32),
    mesh=_SC_MESH,
    scratch_types=_SEG_SCRATCH,
)()

_degcount = functools.partial(
    pl.kernel,
    functools.partial(_segsum_body, False),
    out_type=jax.ShapeDtypeStruct((NC, NPAD, DW), _F32),
    mesh=_SC_MESH,
    scratch_types=_SEG_SCRATCH,
)()


@functools.partial(
    pl.kernel,
    out_type=(jax.ShapeDtypeStruct((EPAD, DW), _F32),
              jax.ShapeDtypeStruct((EPAD, DW), _F32)),
    mesh=_SC_MESH,
    scratch_types=[
        pltpu.VMEM((NCHUNK, CH), jnp.int32),
        pltpu.VMEM((NCHUNK, CH), jnp.int32),
        pltpu.VMEM((CH, DW), _F32),
        pltpu.VMEM((CH, DW), _F32),
        pltpu.SemaphoreType.DMA,
        pltpu.SemaphoreType.DMA,
    ],
)
def _edge_gather(ab_hbm, src_hbm, dst_hbm, g1_hbm, g2_hbm,
                 sidx, didx, rows_a, rows_b, sem_a, sem_b):
    c = lax.axis_index("c")
    s = lax.axis_index("s")
    wid = s * NC + c
    ebase = wid * EPW

    pltpu.sync_copy(src_hbm.at[wid], sidx)
    pltpu.sync_copy(dst_hbm.at[wid], didx)

    def chunk(j, _):
        cp_a = pltpu.async_copy(ab_hbm.at[sidx.at[j]], rows_a, sem_a)
        cp_b = pltpu.async_copy(ab_hbm.at[didx.at[j]], rows_b, sem_b)
        cp_a.wait()
        pltpu.sync_copy(rows_a, g1_hbm.at[pl.ds(ebase + j * CH, CH)])
        cp_b.wait()
        pltpu.sync_copy(rows_b, g2_hbm.at[pl.ds(ebase + j * CH, CH)])
        return 0

    lax.fori_loop(0, NCHUNK, chunk, 0)


_EB = 8000


def kernel(x, edge_index, Wl1, bl1, Wr1, Wl2, bl2, Wr2, Wc1, bc1, Wc2, bc2):
    npadedge = EPAD - E
    src3 = jnp.concatenate(
        [edge_index[0], jnp.zeros((npadedge,), jnp.int32)]).reshape(
            NW, NCHUNK, CH)
    dst3 = jnp.concatenate(
        [edge_index[1], jnp.full((npadedge,), NPAD - 1, jnp.int32)]).reshape(
            NW, NCHUNK, CH)

    cnt = _degcount(jnp.zeros((N, DW), _F32), src3, dst3)

    p1 = pl.pallas_call(
        _tc1_body,
        out_shape=jax.ShapeDtypeStruct((N, DW), _F32),
    )(x, Wl1.T)

    s1 = _segsum(p1, src3, dst3)

    h1, p2, rinv = pl.pallas_call(
        _tc2_body,
        out_shape=(jax.ShapeDtypeStruct((N, D_HID), _F32),
                   jax.ShapeDtypeStruct((N, DW), _F32),
                   jax.ShapeDtypeStruct((N, 1), _F32)),
    )(cnt, s1, x, Wr1.T, bl1.reshape(1, D_HID), Wl2.T)

    s2 = _segsum(p2, src3, dst3)

    ab = pl.pallas_call(
        _tc3_body,
        out_shape=jax.ShapeDtypeStruct((N, DW), _F32),
    )(s2, rinv, h1, Wr2.T, bl2.reshape(1, D_OUT),
      Wc1[:, :D_OUT].T, Wc1[:, D_OUT:].T, bc1.reshape(1, 64))

    g1, g2 = _edge_gather(ab, src3, dst3)

    out = pl.pallas_call(
        _tc4_body,
        grid=(E // _EB,),
        in_specs=[
            pl.BlockSpec((_EB, DW), lambda i: (i, 0)),
            pl.BlockSpec((_EB, DW), lambda i: (i, 0)),
            pl.BlockSpec((D_OUT, 2), lambda i: (0, 0)),
            pl.BlockSpec((1, 2), lambda i: (0, 0)),
        ],
        out_specs=pl.BlockSpec((_EB, 2), lambda i: (i, 0)),
        out_shape=jax.ShapeDtypeStruct((E, 2), _F32),
    )(g1, g2, Wc2.T, bc2.reshape(1, 2))

    return out


# speedup vs baseline: 2.7317x; 1.2140x over previous
import functools

import jax
import jax.numpy as jnp
from jax import lax
from jax.experimental import pallas as pl
from jax.experimental.pallas import tpu as pltpu
from jax.experimental.pallas import tpu_sc as plsc

N = 10000
E = 320000
D_IN = 128
D_HID = 128
D_OUT = 64
DW = 128

NC = 2
NS = 16
NW = NC * NS
CH = 128
NCHUNK = 80
EPW = NCHUNK * CH
RING = 48
EPAD = NW * EPW
NPAD = 10240
RSTRIPE = NPAD // NS
OB = CH
NPAIR = NCHUNK // 2

_F32 = jnp.float32
_HIGH = jax.lax.Precision.HIGHEST


def _dot(a, b):
    return jnp.dot(a, b, preferred_element_type=_F32, precision=_HIGH)


def _tc1_body(x_ref, wl1t_ref, o_ref):
    o_ref[...] = _dot(x_ref[...], wl1t_ref[...])


def _tc2_body(cnt_ref, s1_ref, x_ref, wr1t_ref, bl1_ref, wl2t_ref,
              h1_ref, p2_ref, rinv_ref):
    cnt = (cnt_ref[0] + cnt_ref[1])[:N, :1]
    rinv = 1.0 / jnp.maximum(cnt, 1.0)
    s = (s1_ref[0] + s1_ref[1])[:N]
    mean = s * rinv
    h1 = jnp.maximum(mean + bl1_ref[...] + _dot(x_ref[...], wr1t_ref[...]), 0.0)
    h1_ref[...] = h1
    p2_ref[:, :D_OUT] = _dot(h1, wl2t_ref[...])
    p2_ref[:, D_OUT:] = jnp.zeros((N, DW - D_OUT), _F32)
    rinv_ref[...] = rinv


def _tc3_body(s2_ref, rinv_ref, h1_ref, wr2t_ref, bl2_ref, w1lt_ref,
              w1rt_ref, bc1_ref, ab_ref):
    s = (s2_ref[0] + s2_ref[1])[:N, :D_OUT]
    mean = s * rinv_ref[...]
    h2 = jnp.maximum(mean + bl2_ref[...] + _dot(h1_ref[...], wr2t_ref[...]),
                     0.0)
    ab_ref[:, :D_OUT] = _dot(h2, w1lt_ref[...]) + bc1_ref[...]
    ab_ref[:, D_OUT:] = _dot(h2, w1rt_ref[...])


def _tc4_body(g1_ref, g2_ref, w2t_ref, bc2_ref, o_ref):
    z = jnp.maximum(g1_ref[:, :D_OUT] + g2_ref[:, D_OUT:], 0.0)
    o_ref[...] = _dot(z, w2t_ref[...]) + bc2_ref[...]


_SC_MESH = plsc.VectorSubcoreMesh(
    core_axis_name="c", subcore_axis_name="s", num_cores=NC, num_subcores=NS)


def _fill_vmem(buf, rows, value):
    vec = jnp.full((16,), value, _F32)
    per_row = DW // 16

    def body(i, _):
        buf[i // per_row, pl.ds((i % per_row) * 16, 16)] = vec
        return 0

    lax.fori_loop(0, rows * per_row, body, 0)


def _stripe_init_and_finish(which, rows0, acc, out_hbm, c, s):
    tb = s * RSTRIPE
    if which == "init":
        _fill_vmem(rows0, OB, 0.0)
        for k in range(RSTRIPE // OB):
            pltpu.sync_copy(rows0, acc.at[pl.ds(tb + k * OB, OB)])
    else:
        for k in range(RSTRIPE // OB):
            pltpu.sync_copy(acc.at[pl.ds(tb + k * OB, OB)], rows0)
            pltpu.sync_copy(rows0, out_hbm.at[c, pl.ds(tb + k * OB, OB)])


def _segsum_body(p_hbm, src_hbm, dst_hbm, out_hbm,
                 sidx, didx, rows0, rows1, acc, sg0, sg1, ss0, ss1):
    c = lax.axis_index("c")
    s = lax.axis_index("s")
    wid = s * NC + c
    bufs = (rows0, rows1)
    gsems = (sg0, sg1)
    ssems = (ss0, ss1)

    _stripe_init_and_finish("init", rows0, acc, out_hbm, c, s)

    pltpu.sync_copy(src_hbm.at[wid, pl.ds(0, RING)], sidx)
    pltpu.sync_copy(dst_hbm.at[wid, pl.ds(0, RING)], didx)

    plsc.subcore_barrier()

    for b in range(2):
        pltpu.async_copy(p_hbm.at[sidx.at[b]], bufs[b], gsems[b])

    def pair(k, _):
        for b in range(2):
            j = 2 * k + b
            slot = lax.rem(j, RING)
            pltpu.make_async_copy(
                p_hbm.at[sidx.at[slot]], bufs[b], gsems[b]).wait()
            cp = pltpu.async_copy(
                bufs[b], acc.at[didx.at[slot]], ssems[b], add=True)
            cp.wait()

            if b == 0:
                @pl.when((lax.rem(k, 4) == 0) & (k >= 4) & (k <= 16))
                def _():
                    hbase = pl.multiple_of(j + RING - 8, 8)
                    rbase = pl.multiple_of(lax.rem(j + RING - 8, RING), 8)
                    pltpu.sync_copy(
                        src_hbm.at[wid, pl.ds(hbase, 8)],
                        sidx.at[pl.ds(rbase, 8)])
                    pltpu.sync_copy(
                        dst_hbm.at[wid, pl.ds(hbase, 8)],
                        didx.at[pl.ds(rbase, 8)])

            @pl.when(k < NPAIR - 1)
            def _():
                pltpu.async_copy(
                    p_hbm.at[sidx.at[lax.rem(j + 2, RING)]],
                    bufs[b], gsems[b])
        return 0

    lax.fori_loop(0, NPAIR, pair, 0)

    plsc.subcore_barrier()
    _stripe_init_and_finish("out", rows0, acc, out_hbm, c, s)


def _degcount_body(dst_hbm, out_hbm, didx, rows0, acc, ss0, ss1):
    c = lax.axis_index("c")
    s = lax.axis_index("s")
    wid = s * NC + c
    ssems = (ss0, ss1)

    _stripe_init_and_finish("init", rows0, acc, out_hbm, c, s)
    _fill_vmem(rows0, CH, 1.0)
    pltpu.sync_copy(dst_hbm.at[wid], didx)

    plsc.subcore_barrier()

    def pair(k, _):
        for b in range(2):
            j = 2 * k + b
            pltpu.async_copy(rows0, acc.at[didx.at[j]], ssems[b], add=True)
        for b in range(2):
            pltpu.make_async_copy(
                rows0, acc.at[didx.at[2 * k + b]], ssems[b]).wait()
        return 0

    lax.fori_loop(0, NPAIR, pair, 0)

    plsc.subcore_barrier()
    _stripe_init_and_finish("out", rows0, acc, out_hbm, c, s)


_segsum = functools.partial(
    pl.kernel,
    _segsum_body,
    out_type=jax.ShapeDtypeStruct((NC, NPAD, DW), _F32),
    mesh=_SC_MESH,
    scratch_types=[
        pltpu.VMEM((RING, CH), jnp.int32),
        pltpu.VMEM((RING, CH), jnp.int32),
        pltpu.VMEM((CH, DW), _F32),
        pltpu.VMEM((CH, DW), _F32),
        pltpu.VMEM_SHARED((NPAD, DW), _F32),
        pltpu.SemaphoreType.DMA,
        pltpu.SemaphoreType.DMA,
        pltpu.SemaphoreType.DMA,
        pltpu.SemaphoreType.DMA,
    ],
)()

_degcount = functools.partial(
    pl.kernel,
    _degcount_body,
    out_type=jax.ShapeDtypeStruct((NC, NPAD, DW), _F32),
    mesh=_SC_MESH,
    scratch_types=[
        pltpu.VMEM((NCHUNK, CH), jnp.int32),
        pltpu.VMEM((CH, DW), _F32),
        pltpu.VMEM_SHARED((NPAD, DW), _F32),
        pltpu.SemaphoreType.DMA,
        pltpu.SemaphoreType.DMA,
    ],
)()


@functools.partial(
    pl.kernel,
    out_type=(jax.ShapeDtypeStruct((EPAD, DW), _F32),
              jax.ShapeDtypeStruct((EPAD, DW), _F32)),
    mesh=_SC_MESH,
    scratch_types=[
        pltpu.VMEM((NCHUNK, CH), jnp.int32),
        pltpu.VMEM((NCHUNK, CH), jnp.int32),
        pltpu.VMEM((CH, DW), _F32),
        pltpu.VMEM((CH, DW), _F32),
        pltpu.VMEM((CH, DW), _F32),
        pltpu.VMEM((CH, DW), _F32),
        [pltpu.SemaphoreType.DMA] * 8,
    ],
)
def _edge_gather(ab_hbm, src_hbm, dst_hbm, g1_hbm, g2_hbm,
                 sidx, didx, a0, a1, b0, b1, sems):
    c = lax.axis_index("c")
    s = lax.axis_index("s")
    wid = s * NC + c
    ebase = wid * EPW
    abufs = (a0, a1)
    bbufs = (b0, b1)
    ga, gb, wa, wb = sems[0:2], sems[2:4], sems[4:6], sems[6:8]

    pltpu.sync_copy(src_hbm.at[wid], sidx)
    pltpu.sync_copy(dst_hbm.at[wid], didx)

    for b in range(2):
        pltpu.async_copy(ab_hbm.at[sidx.at[b]], abufs[b], ga[b])
        pltpu.async_copy(ab_hbm.at[didx.at[b]], bbufs[b], gb[b])

    def pair(k, _):
        for b in range(2):
            j = 2 * k + b
            dst_rows = pl.ds(ebase + j * CH, CH)
            pltpu.make_async_copy(
                ab_hbm.at[sidx.at[j]], abufs[b], ga[b]).wait()
            pltpu.async_copy(abufs[b], g1_hbm.at[dst_rows], wa[b])
            pltpu.make_async_copy(
                ab_hbm.at[didx.at[j]], bbufs[b], gb[b]).wait()
            pltpu.async_copy(bbufs[b], g2_hbm.at[dst_rows], wb[b])
            pltpu.make_async_copy(
                abufs[b], g1_hbm.at[dst_rows], wa[b]).wait()
            pltpu.make_async_copy(
                bbufs[b], g2_hbm.at[dst_rows], wb[b]).wait()

            @pl.when(k < NPAIR - 1)
            def _():
                pltpu.async_copy(ab_hbm.at[sidx.at[j + 2]], abufs[b], ga[b])
                pltpu.async_copy(ab_hbm.at[didx.at[j + 2]], bbufs[b], gb[b])
        return 0

    lax.fori_loop(0, NPAIR, pair, 0)


_EB = 8000


def kernel(x, edge_index, Wl1, bl1, Wr1, Wl2, bl2, Wr2, Wc1, bc1, Wc2, bc2):
    npadedge = EPAD - E
    src3 = jnp.concatenate(
        [edge_index[0], jnp.zeros((npadedge,), jnp.int32)]).reshape(
            NW, NCHUNK, CH)
    dst3 = jnp.concatenate(
        [edge_index[1], jnp.full((npadedge,), NPAD - 1, jnp.int32)]).reshape(
            NW, NCHUNK, CH)

    cnt = _degcount(dst3)

    p1 = pl.pallas_call(
        _tc1_body,
        out_shape=jax.ShapeDtypeStruct((N, DW), _F32),
    )(x, Wl1.T)

    s1 = _segsum(p1, src3, dst3)

    h1, p2, rinv = pl.pallas_call(
        _tc2_body,
        out_shape=(jax.ShapeDtypeStruct((N, D_HID), _F32),
                   jax.ShapeDtypeStruct((N, DW), _F32),
                   jax.ShapeDtypeStruct((N, 1), _F32)),
    )(cnt, s1, x, Wr1.T, bl1.reshape(1, D_HID), Wl2.T)

    s2 = _segsum(p2, src3, dst3)

    ab = pl.pallas_call(
        _tc3_body,
        out_shape=jax.ShapeDtypeStruct((N, DW), _F32),
    )(s2, rinv, h1, Wr2.T, bl2.reshape(1, D_OUT),
      Wc1[:, :D_OUT].T, Wc1[:, D_OUT:].T, bc1.reshape(1, 64))

    g1, g2 = _edge_gather(ab, src3, dst3)

    out = pl.pallas_call(
        _tc4_body,
        grid=(E // _EB,),
        in_specs=[
            pl.BlockSpec((_EB, DW), lambda i: (i, 0)),
            pl.BlockSpec((_EB, DW), lambda i: (i, 0)),
            pl.BlockSpec((D_OUT, 2), lambda i: (0, 0)),
            pl.BlockSpec((1, 2), lambda i: (0, 0)),
        ],
        out_specs=pl.BlockSpec((_EB, 2), lambda i: (i, 0)),
        out_shape=jax.ShapeDtypeStruct((E, 2), _F32),
    )(g1, g2, Wc2.T, bc2.reshape(1, 2))

    return out


# speedup vs baseline: 2.7429x; 1.0041x over previous
import functools

import jax
import jax.numpy as jnp
from jax import lax
from jax.experimental import pallas as pl
from jax.experimental.pallas import tpu as pltpu
from jax.experimental.pallas import tpu_sc as plsc

N = 10000
E = 320000
D_IN = 128
D_HID = 128
D_OUT = 64
DW = 128

NC = 2
NS = 16
NW = NC * NS
CH = 128
NCHUNK = 80
EPW = NCHUNK * CH
RING = 48
EPAD = NW * EPW
NPAD = 10240
RSTRIPE = NPAD // NS
OB = CH
NPAIR = NCHUNK // 2
TCH = 2 * NCHUNK
NCH0 = 112
NCH1 = TCH - NCH0
SMAX = max(NCH0, NCH1)

_F32 = jnp.float32
_HIGH = jax.lax.Precision.HIGHEST


def _dot(a, b):
    return jnp.dot(a, b, preferred_element_type=_F32, precision=_HIGH)


def _tc1_body(x_ref, wl1t_ref, o_ref):
    o_ref[...] = _dot(x_ref[...], wl1t_ref[...])


def _tc2_body(cnt_ref, s1_ref, x_ref, wr1t_ref, bl1_ref, wl2t_ref,
              h1_ref, p2_ref, rinv_ref):
    cnt = (cnt_ref[0] + cnt_ref[1])[:N, :1]
    rinv = 1.0 / jnp.maximum(cnt, 1.0)
    s = (s1_ref[0] + s1_ref[1])[:N]
    mean = s * rinv
    h1 = jnp.maximum(mean + bl1_ref[...] + _dot(x_ref[...], wr1t_ref[...]), 0.0)
    h1_ref[...] = h1
    p2_ref[:, :D_OUT] = _dot(h1, wl2t_ref[...])
    p2_ref[:, D_OUT:] = jnp.zeros((N, DW - D_OUT), _F32)
    rinv_ref[...] = rinv


def _tc3_body(s2_ref, rinv_ref, h1_ref, wr2t_ref, bl2_ref, w1lt_ref,
              w1rt_ref, bc1_ref, ab_ref):
    s = (s2_ref[0] + s2_ref[1])[:N, :D_OUT]
    mean = s * rinv_ref[...]
    h2 = jnp.maximum(mean + bl2_ref[...] + _dot(h1_ref[...], wr2t_ref[...]),
                     0.0)
    ab_ref[:, :D_OUT] = _dot(h2, w1lt_ref[...]) + bc1_ref[...]
    ab_ref[:, D_OUT:] = _dot(h2, w1rt_ref[...])


def _tc4_body(g1_ref, g2_ref, w2t_ref, bc2_ref, o_ref):
    z = jnp.maximum(g1_ref[:, :D_OUT] + g2_ref[:, D_OUT:], 0.0)
    o_ref[...] = _dot(z, w2t_ref[...]) + bc2_ref[...]


_SC_MESH = plsc.VectorSubcoreMesh(
    core_axis_name="c", subcore_axis_name="s", num_cores=NC, num_subcores=NS)


def _fill_vmem(buf, rows, value):
    vec = jnp.full((16,), value, _F32)
    per_row = DW // 16

    def body(i, _):
        buf[i // per_row, pl.ds((i % per_row) * 16, 16)] = vec
        return 0

    lax.fori_loop(0, rows * per_row, body, 0)


def _stripe_init_and_finish(which, rows0, acc, out_hbm, c, s):
    tb = s * RSTRIPE
    if which == "init":
        _fill_vmem(rows0, OB, 0.0)
        for k in range(RSTRIPE // OB):
            pltpu.sync_copy(rows0, acc.at[pl.ds(tb + k * OB, OB)])
    else:
        for k in range(RSTRIPE // OB):
            pltpu.sync_copy(acc.at[pl.ds(tb + k * OB, OB)], rows0)
            pltpu.sync_copy(rows0, out_hbm.at[c, pl.ds(tb + k * OB, OB)])


def _segsum_body(p_hbm, src_hbm, dst_hbm, out_hbm,
                 sidx, didx, rows0, rows1, acc, sg0, sg1, ss0, ss1):
    c = lax.axis_index("c")
    s = lax.axis_index("s")
    bufs = (rows0, rows1)
    gsems = (sg0, sg1)
    ssems = (ss0, ss1)
    tc = jnp.where(c == 0, NCH0, NCH1)
    cstart = pl.multiple_of(jnp.where(c == 0, 0, NCH0), 8)
    npair_c = tc // 2

    _stripe_init_and_finish("init", rows0, acc, out_hbm, c, s)

    pltpu.sync_copy(src_hbm.at[s, pl.ds(cstart, RING)], sidx)
    pltpu.sync_copy(dst_hbm.at[s, pl.ds(cstart, RING)], didx)

    plsc.subcore_barrier()

    for b in range(2):
        pltpu.async_copy(p_hbm.at[sidx.at[b]], bufs[b], gsems[b])

    def pair(k, _):
        for b in range(2):
            j = 2 * k + b
            slot = lax.rem(j, RING)
            pltpu.make_async_copy(
                p_hbm.at[sidx.at[slot]], bufs[b], gsems[b]).wait()
            cp = pltpu.async_copy(
                bufs[b], acc.at[didx.at[slot]], ssems[b], add=True)
            cp.wait()

            if b == 0:
                @pl.when((lax.rem(k, 4) == 0) & (j >= 8)
                         & (j + RING - 8 <= tc - 8))
                def _():
                    hbase = pl.multiple_of(
                        cstart + j + RING - 8, 8)
                    rbase = pl.multiple_of(lax.rem(j + RING - 8, RING), 8)
                    pltpu.sync_copy(
                        src_hbm.at[s, pl.ds(hbase, 8)],
                        sidx.at[pl.ds(rbase, 8)])
                    pltpu.sync_copy(
                        dst_hbm.at[s, pl.ds(hbase, 8)],
                        didx.at[pl.ds(rbase, 8)])

            @pl.when(k < npair_c - 1)
            def _():
                pltpu.async_copy(
                    p_hbm.at[sidx.at[lax.rem(j + 2, RING)]],
                    bufs[b], gsems[b])
        return 0

    lax.fori_loop(0, npair_c, pair, 0)

    plsc.subcore_barrier()
    _stripe_init_and_finish("out", rows0, acc, out_hbm, c, s)


def _degcount_body(dst_hbm, out_hbm, didx, rows0, acc, ss0, ss1):
    c = lax.axis_index("c")
    s = lax.axis_index("s")
    ssems = (ss0, ss1)

    _stripe_init_and_finish("init", rows0, acc, out_hbm, c, s)
    _fill_vmem(rows0, CH, 1.0)
    pltpu.sync_copy(
        dst_hbm.at[s, pl.ds(pl.multiple_of(c * NCHUNK, 8), NCHUNK)], didx)

    plsc.subcore_barrier()

    def pair(k, _):
        for b in range(2):
            j = 2 * k + b
            pltpu.async_copy(rows0, acc.at[didx.at[j]], ssems[b], add=True)
        for b in range(2):
            pltpu.make_async_copy(
                rows0, acc.at[didx.at[2 * k + b]], ssems[b]).wait()
        return 0

    lax.fori_loop(0, NPAIR, pair, 0)

    plsc.subcore_barrier()
    _stripe_init_and_finish("out", rows0, acc, out_hbm, c, s)


_segsum = functools.partial(
    pl.kernel,
    _segsum_body,
    out_type=jax.ShapeDtypeStruct((NC, NPAD, DW), _F32),
    mesh=_SC_MESH,
    scratch_types=[
        pltpu.VMEM((RING, CH), jnp.int32),
        pltpu.VMEM((RING, CH), jnp.int32),
        pltpu.VMEM((CH, DW), _F32),
        pltpu.VMEM((CH, DW), _F32),
        pltpu.VMEM_SHARED((NPAD, DW), _F32),
        pltpu.SemaphoreType.DMA,
        pltpu.SemaphoreType.DMA,
        pltpu.SemaphoreType.DMA,
        pltpu.SemaphoreType.DMA,
    ],
)()

_degcount = functools.partial(
    pl.kernel,
    _degcount_body,
    out_type=jax.ShapeDtypeStruct((NC, NPAD, DW), _F32),
    mesh=_SC_MESH,
    scratch_types=[
        pltpu.VMEM((NCHUNK, CH), jnp.int32),
        pltpu.VMEM((CH, DW), _F32),
        pltpu.VMEM_SHARED((NPAD, DW), _F32),
        pltpu.SemaphoreType.DMA,
        pltpu.SemaphoreType.DMA,
    ],
)()


@functools.partial(
    pl.kernel,
    out_type=(jax.ShapeDtypeStruct((EPAD, DW), _F32),
              jax.ShapeDtypeStruct((EPAD, DW), _F32)),
    mesh=_SC_MESH,
    scratch_types=[
        pltpu.VMEM((SMAX, CH), jnp.int32),
        pltpu.VMEM((SMAX, CH), jnp.int32),
        pltpu.VMEM((CH, DW), _F32),
        pltpu.VMEM((CH, DW), _F32),
        pltpu.VMEM((CH, DW), _F32),
        pltpu.VMEM((CH, DW), _F32),
        [pltpu.SemaphoreType.DMA] * 8,
    ],
)
def _edge_gather(ab_hbm, src_hbm, dst_hbm, g1_hbm, g2_hbm,
                 sidx, didx, a0, a1, b0, b1, sems):
    c = lax.axis_index("c")
    s = lax.axis_index("s")
    tc = jnp.where(c == 0, NCH0, NCH1)
    cstart = pl.multiple_of(jnp.where(c == 0, 0, NCH0), 8)
    npair_c = tc // 2
    ebase = (s * TCH + cstart) * CH
    abufs = (a0, a1)
    bbufs = (b0, b1)
    ga, gb, wa, wb = sems[0:2], sems[2:4], sems[4:6], sems[6:8]

    sbase = pl.multiple_of(jnp.where(c == 0, 0, TCH - SMAX), 8)
    joff = cstart - sbase
    pltpu.sync_copy(src_hbm.at[s, pl.ds(sbase, SMAX)], sidx)
    pltpu.sync_copy(dst_hbm.at[s, pl.ds(sbase, SMAX)], didx)

    for b in range(2):
        pltpu.async_copy(ab_hbm.at[sidx.at[joff + b]], abufs[b], ga[b])
        pltpu.async_copy(ab_hbm.at[didx.at[joff + b]], bbufs[b], gb[b])

    def pair(k, _):
        for b in range(2):
            j = 2 * k + b
            dst_rows = pl.ds(ebase + j * CH, CH)
            pltpu.make_async_copy(
                ab_hbm.at[sidx.at[joff + j]], abufs[b], ga[b]).wait()
            pltpu.async_copy(abufs[b], g1_hbm.at[dst_rows], wa[b])
            pltpu.make_async_copy(
                ab_hbm.at[didx.at[joff + j]], bbufs[b], gb[b]).wait()
            pltpu.async_copy(bbufs[b], g2_hbm.at[dst_rows], wb[b])
            pltpu.make_async_copy(
                abufs[b], g1_hbm.at[dst_rows], wa[b]).wait()
            pltpu.make_async_copy(
                bbufs[b], g2_hbm.at[dst_rows], wb[b]).wait()

            @pl.when(k < npair_c - 1)
            def _():
                pltpu.async_copy(
                    ab_hbm.at[sidx.at[joff + j + 2]], abufs[b], ga[b])
                pltpu.async_copy(
                    ab_hbm.at[didx.at[joff + j + 2]], bbufs[b], gb[b])
        return 0

    lax.fori_loop(0, npair_c, pair, 0)


_EB = 8000


def kernel(x, edge_index, Wl1, bl1, Wr1, Wl2, bl2, Wr2, Wc1, bc1, Wc2, bc2):
    npadedge = EPAD - E
    src3 = jnp.concatenate(
        [edge_index[0], jnp.zeros((npadedge,), jnp.int32)]).reshape(
            NS, TCH, CH)
    dst3 = jnp.concatenate(
        [edge_index[1], jnp.full((npadedge,), NPAD - 1, jnp.int32)]).reshape(
            NS, TCH, CH)

    cnt = _degcount(dst3)

    p1 = pl.pallas_call(
        _tc1_body,
        out_shape=jax.ShapeDtypeStruct((N, DW), _F32),
    )(x, Wl1.T)

    s1 = _segsum(p1, src3, dst3)

    h1, p2, rinv = pl.pallas_call(
        _tc2_body,
        out_shape=(jax.ShapeDtypeStruct((N, D_HID), _F32),
                   jax.ShapeDtypeStruct((N, DW), _F32),
                   jax.ShapeDtypeStruct((N, 1), _F32)),
    )(cnt, s1, x, Wr1.T, bl1.reshape(1, D_HID), Wl2.T)

    s2 = _segsum(p2, src3, dst3)

    ab = pl.pallas_call(
        _tc3_body,
        out_shape=jax.ShapeDtypeStruct((N, DW), _F32),
    )(s2, rinv, h1, Wr2.T, bl2.reshape(1, D_OUT),
      Wc1[:, :D_OUT].T, Wc1[:, D_OUT:].T, bc1.reshape(1, 64))

    g1, g2 = _edge_gather(ab, src3, dst3)

    out = pl.pallas_call(
        _tc4_body,
        grid=(E // _EB,),
        in_specs=[
            pl.BlockSpec((_EB, DW), lambda i: (i, 0)),
            pl.BlockSpec((_EB, DW), lambda i: (i, 0)),
            pl.BlockSpec((D_OUT, 2), lambda i: (0, 0)),
            pl.BlockSpec((1, 2), lambda i: (0, 0)),
        ],
        out_specs=pl.BlockSpec((_EB, 2), lambda i: (i, 0)),
        out_shape=jax.ShapeDtypeStruct((E, 2), _F32),
    )(g1, g2, Wc2.T, bc2.reshape(1, 2))

    return out
